# baseline (speedup 1.0000x reference)
import functools
import math

import jax
import jax.numpy as jnp
from jax import lax
from jax.experimental import pallas as pl
from jax.experimental.pallas import tpu as pltpu
from jax.experimental.pallas import tpu_sc as plsc

NC = 2
NS = 16
L = 16

COL_TF = 24
COL_DL = 14
COL_PR = 129
COL_IL = 127
COL_OL = 128
COL_AVG = 16
STRIPS = (
    (3, ((0, 0),)),
    (2, ()),
    (1, ((6, 1),)),
    (16, ((0, 4), (1, 2))),
    (15, ((7, 3),)),
)


@functools.partial(jax.jit, static_argnums=(4,))
def _sc_rank(idx, gf_t, fresh, params, n_docs):
    B = idx.shape[0]
    NW = NC * NS
    b_per_w = B // NW
    s_iter = b_per_w // L
    n_tiles = (n_docs + 127) // 128
    n_pad = n_tiles * 128
    t_hi = n_tiles // NS + 1
    n_hi = n_tiles - (n_tiles // NS) * NS
    chunk_hi = t_hi * 128
    chunk_lo = (t_hi - 1) * 128
    m_buf = chunk_hi

    mesh = plsc.VectorSubcoreMesh(core_axis_name="c", subcore_axis_name="s")

    @functools.partial(
        pl.kernel,
        mesh=mesh,
        out_type=jax.ShapeDtypeStruct((B,), jnp.float32),
        scratch_types=[
            pltpu.VMEM((8, L), jnp.float32),
            pltpu.VMEM((2, 8, m_buf // 2 + 64), jnp.float32),
            pltpu.VMEM((5 * b_per_w,), jnp.int32),
            pltpu.VMEM((5 * b_per_w,), jnp.float32),
            pltpu.VMEM((b_per_w,), jnp.int32),
            pltpu.VMEM((b_per_w,), jnp.float32),
            pltpu.VMEM((b_per_w,), jnp.float32),
            pltpu.VMEM((L,), jnp.float32),
            pltpu.VMEM((NS, L), jnp.float32),
            pltpu.VMEM_SHARED((5 * n_pad,), jnp.float32),
            pltpu.VMEM_SHARED((NS, L), jnp.float32),
            pltpu.SemaphoreType.DMA,
            pltpu.SemaphoreType.DMA,
            pltpu.SemaphoreType.DMA,
        ],
    )
    def k(idx_hbm, gft_hbm, fresh_hbm, par_hbm, out_hbm,
          par_v, strip_v, sidx_v, sval_v, bidx_v, fresh_v, out_v,
          acc_v, part_v, cols_sh, shared_v, sem_m, sem_g, sem_s):
        c = lax.axis_index("c")
        s = lax.axis_index("s")
        wid = c * NS + s
        base = wid * b_per_w
        lane = lax.iota(jnp.int32, L)

        m_off = jnp.where(s < n_hi, s * chunk_hi,
                          n_hi * chunk_hi + (s - n_hi) * chunk_lo)
        m_len = jnp.where(s < n_hi, chunk_hi, chunk_lo)
        m_iters = m_len // L

        h0_len = jnp.where(s < n_hi, chunk_hi // 2 + 64, chunk_lo // 2)
        h1_len = m_len - h0_len
        halves = [(0, h0_len), (h0_len, h1_len)]
        sems = (sem_m, sem_g)
        steps = []
        for strip, extracts in STRIPS:
            for h_off, h_len in halves:
                steps.append((strip, extracts, h_off, h_len))
        copies = [
            pltpu.make_async_copy(
                gft_hbm.at[pl.ds(strip * 8, 8), pl.ds(m_off + h_off, h_len)],
                strip_v.at[i % 2, :, pl.ds(0, h_len)], sems[i % 2])
            for i, (strip, _, h_off, h_len) in enumerate(steps)
        ]
        copies[0].start()
        copies[1].start()

        pltpu.sync_copy(idx_hbm.at[pl.ds(base, b_per_w)], bidx_v)
        pltpu.sync_copy(fresh_hbm.at[pl.ds(base, b_per_w)], fresh_v)
        pltpu.sync_copy(par_hbm, par_v)

        def sbody(i, carry):
            v = bidx_v[pl.ds(i * L, L)]
            for j in range(5):
                sidx_v[pl.ds(j * b_per_w + i * L, L)] = v + j * n_pad
            return carry

        lax.fori_loop(0, s_iter, sbody, 0)

        acc = jnp.zeros((L,), jnp.float32)
        for i, (strip, extracts, h_off, h_len) in enumerate(steps):
            copies[i].wait()
            buf = strip_v.at[i % 2]
            for sub, slot in extracts:
                pltpu.sync_copy(
                    buf.at[sub, pl.ds(0, h_len)],
                    cols_sh.at[pl.ds(slot * n_pad + m_off + h_off, h_len)])
            if strip == COL_AVG // 8:
                sub = COL_AVG % 8
                h_iters = h_len // L
                h_base = m_off + h_off

                def rbody(ii, a, buf=buf, h_base=h_base):
                    v = buf[sub, pl.ds(ii * L, L)]
                    ok = (h_base + ii * L + lane) < n_docs
                    return a + jnp.where(ok, v, 0.0)

                acc = lax.fori_loop(0, h_iters, rbody, acc)
            if i + 2 < len(steps):
                copies[i + 2].start()

        acc_v[...] = acc
        pltpu.sync_copy(acc_v, shared_v.at[s])
        plsc.subcore_barrier()

        pltpu.sync_copy(shared_v, part_v)
        tot = part_v[0]
        for j in range(1, NS):
            tot = tot + part_v[j]
        tot_s = tot[0]
        for j in range(1, L):
            tot_s = tot_s + tot[j]
        inv_avg = jnp.full((L,), float(n_docs), jnp.float32) / jnp.broadcast_to(tot_s, (L,))

        gcopy = pltpu.make_async_copy(cols_sh.at[sidx_v], sval_v, sem_s)
        gcopy.start()

        a_c = par_v[0]
        k1_1mb = par_v[1]
        k1b = par_v[2]
        pr_c = par_v[3]
        il_c = par_v[4]
        ol_c = par_v[5]
        f_c = par_v[6]
        gcopy.wait()

        def cbody(i, carry):
            tf = sval_v[pl.ds(i * L, L)]
            dl = sval_v[pl.ds(b_per_w + i * L, L)]
            prv = sval_v[pl.ds(2 * b_per_w + i * L, L)]
            ilv = sval_v[pl.ds(3 * b_per_w + i * L, L)]
            olv = sval_v[pl.ds(4 * b_per_w + i * L, L)]
            fu = fresh_v[pl.ds(i * L, L)]
            denom = tf + k1_1mb + k1b * (dl * inv_avg)
            score = a_c * tf / denom + pr_c * prv + il_c * ilv + ol_c * olv + f_c * fu
            out_v[pl.ds(i * L, L)] = score
            return carry

        lax.fori_loop(0, s_iter, cbody, 0)
        pltpu.sync_copy(out_v, out_hbm.at[pl.ds(base, b_per_w)])

    return k(idx, gf_t, fresh, params)


def kernel(batch_indices, global_features, fresh_u, bm25_k1, bm25_b,
           bm25_weight, page_rank, in_link, out_link, freshness):
    n_docs, _ = global_features.shape
    idf = math.log(0.5 / (n_docs + 0.5) + 1.0)
    a_c = bm25_weight * idf * (bm25_k1 + 1.0)
    params = jnp.stack([
        a_c.astype(jnp.float32),
        (bm25_k1 * (1.0 - bm25_b)).astype(jnp.float32),
        (bm25_k1 * bm25_b).astype(jnp.float32),
        page_rank.astype(jnp.float32),
        in_link.astype(jnp.float32),
        out_link.astype(jnp.float32),
        freshness.astype(jnp.float32),
        jnp.zeros((), jnp.float32),
    ])
    params = jnp.broadcast_to(params[:, None], (8, L))
    out = _sc_rank(batch_indices.astype(jnp.int32), global_features.T,
                   fresh_u.astype(jnp.float32), params, n_docs)
    return out[:, None]
